# baseline (device time: 78823 ns/iter reference)
import jax
import jax.numpy as jnp
from jax import lax
from jax.experimental import pallas as pl
from jax.experimental.pallas import tpu as pltpu

N_DEV = 4
N_COLS_GLOBAL = 4096
EPS = 1e-5
CHUNK = 512


def kernel(x, gamma):
    m, n = x.shape
    nc = m // CHUNK
    g2 = gamma.reshape(1, n)

    def body(x_ref, g_ref, out_ref, xv_ref, part_ref, comm_ref, inv_ref,
             send_sems, recv_sems):
        c = pl.program_id(0)
        my = lax.axis_index("i")

        @pl.when(c == 0)
        def _():
            barrier_sem = pltpu.get_barrier_semaphore()
            for j in range(1, N_DEV):
                peer = (my + j) % N_DEV
                pl.semaphore_signal(
                    barrier_sem, inc=1,
                    device_id=(peer,), device_id_type=pl.DeviceIdType.MESH,
                )
            pl.semaphore_wait(barrier_sem, N_DEV - 1)

        @pl.when(c < nc)
        def _():
            xv = x_ref[...]
            xv_ref[pl.ds(c * CHUNK, CHUNK), :] = xv
            part_ref[pl.ds(c * CHUNK, CHUNK), :] = jnp.sum(
                xv * xv, axis=1, keepdims=True
            )

        def mk_rdma(j):
            peer = (my + j) % N_DEV
            return pltpu.make_async_remote_copy(
                src_ref=part_ref,
                dst_ref=comm_ref.at[j - 1],
                send_sem=send_sems.at[j - 1],
                recv_sem=recv_sems.at[j - 1],
                device_id=(peer,),
                device_id_type=pl.DeviceIdType.MESH,
            )

        @pl.when(c == nc - 1)
        def _():
            for j in range(1, N_DEV):
                mk_rdma(j).start()

        @pl.when(c == nc)
        def _():
            for j in range(1, N_DEV):
                r = mk_rdma(j)
                r.wait_recv()
                r.wait_send()
            total = part_ref[...] + comm_ref[0] + comm_ref[1] + comm_ref[2]
            inv_ref[...] = lax.rsqrt(total * (1.0 / N_COLS_GLOBAL) + EPS)

        @pl.when(c >= nc)
        def _():
            cc = c - nc
            xv = xv_ref[pl.ds(cc * CHUNK, CHUNK), :]
            out_ref[...] = xv * inv_ref[pl.ds(cc * CHUNK, CHUNK), :] * g_ref[...]

    return pl.pallas_call(
        body,
        grid=(2 * nc,),
        out_shape=jax.ShapeDtypeStruct((m, n), jnp.float32),
        in_specs=[
            pl.BlockSpec((CHUNK, n), lambda c: (jnp.minimum(c, nc - 1), 0)),
            pl.BlockSpec((1, n), lambda c: (0, 0)),
        ],
        out_specs=pl.BlockSpec((CHUNK, n), lambda c: (jnp.maximum(c - nc, 0), 0)),
        scratch_shapes=[
            pltpu.VMEM((m, n), jnp.float32),
            pltpu.VMEM((m, 1), jnp.float32),
            pltpu.VMEM((3, m, 1), jnp.float32),
            pltpu.VMEM((m, 1), jnp.float32),
            pltpu.SemaphoreType.DMA((3,)),
            pltpu.SemaphoreType.DMA((3,)),
        ],
        compiler_params=pltpu.CompilerParams(
            collective_id=0,
            vmem_limit_bytes=64 * 1024 * 1024,
        ),
    )(x, g2)
